# baseline (speedup 1.0000x reference)
import functools

import jax
import jax.numpy as jnp
from jax.experimental import pallas as pl


def _pos_body(row_ref, col_ref, depth_ref, out_ref, *, hb, w):
    di = pl.program_id(0)
    x = col_ref[0:w, :]
    y = row_ref[...]
    z = depth_ref[pl.ds(di, 1), :]
    xb = jnp.broadcast_to(x[None, :, :], (hb, w, 64))
    yb = jnp.broadcast_to(y[:, None, :], (hb, w, 64))
    zb = jnp.broadcast_to(z[None, :, :], (hb, w, 64))
    out_ref[...] = jnp.concatenate([xb, yb, zb], axis=-1)[None]


def kernel(scan, row_weight, col_weight, depth_weight):
    d, em, h, w = scan.shape
    hb = 32
    n_h = h // hb
    body = functools.partial(_pos_body, hb=hb, w=w)
    out = pl.pallas_call(
        body,
        grid=(d, n_h),
        in_specs=[
            pl.BlockSpec((hb, 64), lambda di, hi: (hi, 0)),
            pl.BlockSpec((256, 64), lambda di, hi: (0, 0)),
            pl.BlockSpec((40, 64), lambda di, hi: (0, 0)),
        ],
        out_specs=pl.BlockSpec((1, hb, w, 192), lambda di, hi: (di, hi, 0, 0)),
        out_shape=jax.ShapeDtypeStruct((d, h, w, 192), jnp.float32),
    )(row_weight, col_weight, depth_weight)
    return out
